# baseline (speedup 1.0000x reference)
import functools

import jax
import jax.numpy as jnp
from jax import lax
from jax.experimental import pallas as pl
from jax.experimental.pallas import tpu as pltpu
from jax.experimental.pallas import tpu_sc as plsc

NUM_RELATIONS = 1000
EMBED_DIM = 64
HIDDEN = 128
B, S = 1024, 20
N = B * S
NC, NS = 2, 16
NW = NC * NS
ROWS_PER_W = N // NW
R_CHUNK = 4
IDX_PER_CHUNK = R_CHUNK * S
N_CHUNKS = ROWS_PER_W // R_CHUNK
NBUF = 8


def _sc_gather_mean(idx2d, table):
    mesh = plsc.VectorSubcoreMesh(core_axis_name="c", subcore_axis_name="s")

    @functools.partial(
        pl.kernel,
        out_type=jax.ShapeDtypeStruct((N, EMBED_DIM), jnp.float32),
        mesh=mesh,
        scratch_types=[
            pltpu.VMEM((N_CHUNKS, IDX_PER_CHUNK), jnp.int32),
            [pltpu.VMEM((IDX_PER_CHUNK, EMBED_DIM), jnp.float32)] * NBUF,
            [pltpu.VMEM((R_CHUNK, EMBED_DIM), jnp.float32)] * NBUF,
            [pltpu.SemaphoreType.DMA] * NBUF,
            [pltpu.SemaphoreType.DMA] * NBUF,
        ],
        compiler_params=pltpu.CompilerParams(use_tc_tiling_on_sc=False),
    )
    def k(idx_hbm, table_hbm, agg_hbm, idx_v, rows_v, out_v, gsem, osem):
        wid = lax.axis_index("s") * NC + lax.axis_index("c")
        chunk0 = wid * N_CHUNKS

        pltpu.sync_copy(idx_hbm.at[pl.ds(chunk0 * 1, N_CHUNKS)], idx_v)

        def gather(g, b):
            pltpu.make_async_copy(
                table_hbm.at[idx_v.at[g]], rows_v[b], gsem[b]
            ).start()

        def gather_wait(b):
            pltpu.make_async_copy(
                table_hbm.at[idx_v.at[0]], rows_v[b], gsem[b]
            ).wait()

        def out_start(g, b):
            pltpu.make_async_copy(
                out_v[b], agg_hbm.at[pl.ds((chunk0 + g) * R_CHUNK, R_CHUNK)],
                osem[b],
            ).start()

        def out_wait(b):
            pltpu.make_async_copy(
                out_v[b], agg_hbm.at[pl.ds(0, R_CHUNK)], osem[b]
            ).wait()

        for b in range(NBUF):
            gather(b, b)

        def outer(t, carry):
            for b in range(NBUF):
                g = t * NBUF + b
                gather_wait(b)
                @pl.when(g >= NBUF)
                def _():
                    out_wait(b)

                rv, ov = rows_v[b], out_v[b]
                for rr in range(R_CHUNK):
                    for c in range(EMBED_DIM // 16):
                        acc = rv[rr * S, pl.ds(c * 16, 16)]
                        for j in range(1, S):
                            acc = acc + rv[rr * S + j, pl.ds(c * 16, 16)]
                        ov[rr, pl.ds(c * 16, 16)] = acc * (1.0 / S)
                out_start(g, b)

                @pl.when(g + NBUF < N_CHUNKS)
                def _():
                    gather(g + NBUF, b)

            return carry

        lax.fori_loop(0, N_CHUNKS // NBUF, outer, None)
        for b in range(NBUF):
            out_wait(b)

    return k(idx2d, table)


def _mlp(agg, W1, b1, W2, b2):
    ROWS_BLK = 2048

    def body(a_ref, w1_ref, b1_ref, w2_ref, b2_ref, o_ref):
        h = jnp.dot(a_ref[...], w1_ref[...], preferred_element_type=jnp.float32)
        h = jnp.maximum(h + b1_ref[...], 0.0)
        o_ref[...] = (
            jnp.dot(h, w2_ref[...], preferred_element_type=jnp.float32)
            + b2_ref[...]
        )

    return pl.pallas_call(
        body,
        grid=(N // ROWS_BLK,),
        in_specs=[
            pl.BlockSpec((ROWS_BLK, EMBED_DIM), lambda i: (i, 0)),
            pl.BlockSpec((EMBED_DIM, HIDDEN), lambda i: (0, 0)),
            pl.BlockSpec((1, HIDDEN), lambda i: (0, 0)),
            pl.BlockSpec((HIDDEN, HIDDEN), lambda i: (0, 0)),
            pl.BlockSpec((1, HIDDEN), lambda i: (0, 0)),
        ],
        out_specs=pl.BlockSpec((ROWS_BLK, HIDDEN), lambda i: (i, 0)),
        out_shape=jax.ShapeDtypeStruct((N, HIDDEN), jnp.float32),
    )(agg, W1, b1, W2, b2)


def kernel(kg_spatial_matrix, rel_table, W1, b1, W2, b2):
    table = rel_table.at[0].set(0.0)
    idx2d = kg_spatial_matrix.reshape(N // R_CHUNK, IDX_PER_CHUNK)
    agg = _sc_gather_mean(idx2d, table)
    out = _mlp(agg, W1, b1.reshape(1, HIDDEN), W2, b2.reshape(1, HIDDEN))
    return out.reshape(B, S, HIDDEN)


# speedup vs baseline: 1.2266x; 1.2266x over previous
import functools

import jax
import jax.numpy as jnp
from jax import lax
from jax.experimental import pallas as pl
from jax.experimental.pallas import tpu as pltpu
from jax.experimental.pallas import tpu_sc as plsc

NUM_RELATIONS = 1000
EMBED_DIM = 64
HIDDEN = 128
B, S = 1024, 20
N = B * S
NC, NS = 2, 16
NW = NC * NS
ROWS_PER_W = N // NW
R_CHUNK = 4
IDX_PER_CHUNK = R_CHUNK * S
N_CHUNKS = ROWS_PER_W // R_CHUNK
NBUF = 4


def _sc_gather_mean(idx2d, table):
    mesh = plsc.VectorSubcoreMesh(core_axis_name="c", subcore_axis_name="s")

    @functools.partial(
        pl.kernel,
        out_type=jax.ShapeDtypeStruct((N, EMBED_DIM // 2), jnp.uint32),
        mesh=mesh,
        scratch_types=[
            pltpu.VMEM((N_CHUNKS, IDX_PER_CHUNK), jnp.int32),
            [pltpu.VMEM((IDX_PER_CHUNK, EMBED_DIM // 2), jnp.uint32)] * NBUF,
            [pltpu.VMEM((R_CHUNK, EMBED_DIM // 2), jnp.uint32)] * NBUF,
            [pltpu.SemaphoreType.DMA] * NBUF,
            [pltpu.SemaphoreType.DMA] * NBUF,
        ],
        compiler_params=pltpu.CompilerParams(use_tc_tiling_on_sc=False),
    )
    def k(idx_hbm, table_hbm, agg_hbm, idx_v, rows_v, out_v, gsem, osem):
        wid = lax.axis_index("s") * NC + lax.axis_index("c")
        chunk0 = wid * N_CHUNKS

        pltpu.sync_copy(idx_hbm.at[pl.ds(chunk0 * 1, N_CHUNKS)], idx_v)

        def gather(g, b):
            pltpu.make_async_copy(
                table_hbm.at[idx_v.at[g]], rows_v[b], gsem[b]
            ).start()

        def gather_wait(b):
            pltpu.make_async_copy(
                table_hbm.at[idx_v.at[0]], rows_v[b], gsem[b]
            ).wait()

        def out_start(g, b):
            pltpu.make_async_copy(
                out_v[b], agg_hbm.at[pl.ds((chunk0 + g) * R_CHUNK, R_CHUNK)],
                osem[b],
            ).start()

        def out_wait(b):
            pltpu.make_async_copy(
                out_v[b], agg_hbm.at[pl.ds(0, R_CHUNK)], osem[b]
            ).wait()

        for b in range(NBUF):
            gather(b, b)

        def outer(t, carry):
            for b in range(NBUF):
                g = t * NBUF + b
                gather_wait(b)
                @pl.when(g >= NBUF)
                def _():
                    out_wait(b)

                rv, ov = rows_v[b], out_v[b]
                hi_mask = jnp.uint32(0xFFFF0000)
                for rr in range(R_CHUNK):
                    for c in range(EMBED_DIM // 32):
                        acc_a = jnp.zeros((16,), jnp.float32)
                        acc_b = jnp.zeros((16,), jnp.float32)
                        for j in range(S):
                            u = rv[rr * S + j, pl.ds(c * 16, 16)]
                            a = lax.bitcast_convert_type(u << 16, jnp.float32)
                            bb = lax.bitcast_convert_type(u & hi_mask, jnp.float32)
                            acc_a = acc_a + a
                            acc_b = acc_b + bb
                        ua = lax.bitcast_convert_type(acc_a * (1.0 / S), jnp.uint32)
                        ub = lax.bitcast_convert_type(acc_b * (1.0 / S), jnp.uint32)
                        ov[rr, pl.ds(c * 16, 16)] = (ua >> 16) | (ub & hi_mask)
                out_start(g, b)

                @pl.when(g + NBUF < N_CHUNKS)
                def _():
                    gather(g + NBUF, b)

            return carry

        lax.fori_loop(0, N_CHUNKS // NBUF, outer, None)
        for b in range(NBUF):
            out_wait(b)

    return k(idx2d, table)


def _mlp(agg, W1, b1, W2, b2):
    ROWS_BLK = 2048

    def body(a_ref, w1_ref, b1_ref, w2_ref, b2_ref, o_ref):
        a = a_ref[...].astype(jnp.float32)
        h = jnp.dot(a, w1_ref[...], preferred_element_type=jnp.float32)
        h = jnp.maximum(h + b1_ref[...], 0.0)
        o_ref[...] = (
            jnp.dot(h, w2_ref[...], preferred_element_type=jnp.float32)
            + b2_ref[...]
        )

    return pl.pallas_call(
        body,
        grid=(N // ROWS_BLK,),
        in_specs=[
            pl.BlockSpec((ROWS_BLK, EMBED_DIM), lambda i: (i, 0)),
            pl.BlockSpec((EMBED_DIM, HIDDEN), lambda i: (0, 0)),
            pl.BlockSpec((1, HIDDEN), lambda i: (0, 0)),
            pl.BlockSpec((HIDDEN, HIDDEN), lambda i: (0, 0)),
            pl.BlockSpec((1, HIDDEN), lambda i: (0, 0)),
        ],
        out_specs=pl.BlockSpec((ROWS_BLK, HIDDEN), lambda i: (i, 0)),
        out_shape=jax.ShapeDtypeStruct((N, HIDDEN), jnp.float32),
    )(agg, W1, b1, W2, b2)


def kernel(kg_spatial_matrix, rel_table, W1, b1, W2, b2):
    table_bf = rel_table.at[0].set(0.0).astype(jnp.bfloat16)
    table_u32 = jax.lax.bitcast_convert_type(
        table_bf.reshape(NUM_RELATIONS, EMBED_DIM // 2, 2), jnp.uint32
    )
    idx2d = kg_spatial_matrix.reshape(N // R_CHUNK, IDX_PER_CHUNK)
    agg_u32 = _sc_gather_mean(idx2d, table_u32)
    agg = jax.lax.bitcast_convert_type(agg_u32, jnp.bfloat16).reshape(
        N, EMBED_DIM
    )
    out = _mlp(agg, W1, b1.reshape(1, HIDDEN), W2, b2.reshape(1, HIDDEN))
    return out.reshape(B, S, HIDDEN)


# speedup vs baseline: 1.5131x; 1.2335x over previous
import functools

import jax
import jax.numpy as jnp
from jax import lax
from jax.experimental import pallas as pl
from jax.experimental.pallas import tpu as pltpu
from jax.experimental.pallas import tpu_sc as plsc

NUM_RELATIONS = 1000
EMBED_DIM = 64
HIDDEN = 128
B, S = 1024, 20
N = B * S
NC, NS = 2, 16
NW = NC * NS
ROWS_PER_W = N // NW
R_CHUNK = 4
IDX_PER_CHUNK = R_CHUNK * S
N_CHUNKS = ROWS_PER_W // R_CHUNK
NBUF = 4


def _sc_gather_mean(idx2d, table):
    mesh = plsc.VectorSubcoreMesh(core_axis_name="c", subcore_axis_name="s")

    @functools.partial(
        pl.kernel,
        out_type=jax.ShapeDtypeStruct((N, EMBED_DIM // 2), jnp.uint32),
        mesh=mesh,
        scratch_types=[
            pltpu.VMEM((N_CHUNKS, IDX_PER_CHUNK), jnp.int32),
            [pltpu.VMEM((IDX_PER_CHUNK, EMBED_DIM // 2), jnp.uint32)] * NBUF,
            [pltpu.VMEM((R_CHUNK, EMBED_DIM // 2), jnp.uint32)] * NBUF,
            [pltpu.SemaphoreType.DMA] * NBUF,
            [pltpu.SemaphoreType.DMA] * NBUF,
        ],
        compiler_params=pltpu.CompilerParams(use_tc_tiling_on_sc=False),
    )
    def k(idx_hbm, table_hbm, agg_hbm, idx_v, rows_v, out_v, gsem, osem):
        wid = lax.axis_index("s") * NC + lax.axis_index("c")
        chunk0 = wid * N_CHUNKS

        pltpu.sync_copy(idx_hbm.at[pl.ds(chunk0 * 1, N_CHUNKS)], idx_v)

        def gather(g, b):
            pltpu.make_async_copy(
                table_hbm.at[idx_v.at[g]], rows_v[b], gsem[b]
            ).start()

        def gather_wait(b):
            pltpu.make_async_copy(
                table_hbm.at[idx_v.at[0]], rows_v[b], gsem[b]
            ).wait()

        def out_start(g, b):
            pltpu.make_async_copy(
                out_v[b], agg_hbm.at[pl.ds((chunk0 + g) * R_CHUNK, R_CHUNK)],
                osem[b],
            ).start()

        def out_wait(b):
            pltpu.make_async_copy(
                out_v[b], agg_hbm.at[pl.ds(0, R_CHUNK)], osem[b]
            ).wait()

        for b in range(NBUF):
            gather(b, b)

        def outer(t, carry):
            for b in range(NBUF):
                g = t * NBUF + b
                gather_wait(b)
                @pl.when(g >= NBUF)
                def _():
                    out_wait(b)

                rv, ov = rows_v[b], out_v[b]
                hi_mask = jnp.uint32(0xFFFF0000)
                for rr in range(R_CHUNK):
                    for c in range(EMBED_DIM // 32):
                        acc_a = jnp.zeros((16,), jnp.float32)
                        acc_b = jnp.zeros((16,), jnp.float32)
                        for j in range(S):
                            u = rv[rr * S + j, pl.ds(c * 16, 16)]
                            a = lax.bitcast_convert_type(u << 16, jnp.float32)
                            bb = lax.bitcast_convert_type(u & hi_mask, jnp.float32)
                            acc_a = acc_a + a
                            acc_b = acc_b + bb
                        ua = lax.bitcast_convert_type(acc_a * (1.0 / S), jnp.uint32)
                        ub = lax.bitcast_convert_type(acc_b * (1.0 / S), jnp.uint32)
                        ov[rr, pl.ds(c * 16, 16)] = (ua >> 16) | (ub & hi_mask)
                out_start(g, b)

                @pl.when(g + NBUF < N_CHUNKS)
                def _():
                    gather(g + NBUF, b)

            return carry

        lax.fori_loop(0, N_CHUNKS // NBUF, outer, None)
        for b in range(NBUF):
            out_wait(b)

    return k(idx2d, table)


def _mlp(agg_u32, W1e, W1o, b1, W2, b2):
    ROWS_BLK = 2048

    def body(a_ref, w1e_ref, w1o_ref, b1_ref, w2_ref, b2_ref, o_ref):
        u = a_ref[...]
        a_even = lax.bitcast_convert_type(u << 16, jnp.float32)
        a_odd = lax.bitcast_convert_type(u & jnp.uint32(0xFFFF0000),
                                         jnp.float32)
        h = (
            jnp.dot(a_even, w1e_ref[...], preferred_element_type=jnp.float32)
            + jnp.dot(a_odd, w1o_ref[...], preferred_element_type=jnp.float32)
        )
        h = jnp.maximum(h + b1_ref[...], 0.0)
        o_ref[...] = (
            jnp.dot(h, w2_ref[...], preferred_element_type=jnp.float32)
            + b2_ref[...]
        )

    return pl.pallas_call(
        body,
        grid=(N // ROWS_BLK,),
        in_specs=[
            pl.BlockSpec((ROWS_BLK, EMBED_DIM // 2), lambda i: (i, 0)),
            pl.BlockSpec((EMBED_DIM // 2, HIDDEN), lambda i: (0, 0)),
            pl.BlockSpec((EMBED_DIM // 2, HIDDEN), lambda i: (0, 0)),
            pl.BlockSpec((1, HIDDEN), lambda i: (0, 0)),
            pl.BlockSpec((HIDDEN, HIDDEN), lambda i: (0, 0)),
            pl.BlockSpec((1, HIDDEN), lambda i: (0, 0)),
        ],
        out_specs=pl.BlockSpec((ROWS_BLK, HIDDEN), lambda i: (i, 0)),
        out_shape=jax.ShapeDtypeStruct((N, HIDDEN), jnp.float32),
    )(agg_u32, W1e, W1o, b1, W2, b2)


def kernel(kg_spatial_matrix, rel_table, W1, b1, W2, b2):
    table_bf = rel_table.at[0].set(0.0).astype(jnp.bfloat16)
    table_u32 = jax.lax.bitcast_convert_type(
        table_bf.reshape(NUM_RELATIONS, EMBED_DIM // 2, 2), jnp.uint32
    )
    idx2d = kg_spatial_matrix.reshape(N // R_CHUNK, IDX_PER_CHUNK)
    agg_u32 = _sc_gather_mean(idx2d, table_u32)
    out = _mlp(agg_u32, W1[0::2], W1[1::2], b1.reshape(1, HIDDEN), W2,
               b2.reshape(1, HIDDEN))
    return out.reshape(B, S, HIDDEN)


# speedup vs baseline: 1.9512x; 1.2895x over previous
import functools

import jax
import jax.numpy as jnp
from jax import lax
from jax.experimental import pallas as pl
from jax.experimental.pallas import tpu as pltpu
from jax.experimental.pallas import tpu_sc as plsc

NUM_RELATIONS = 1000
EMBED_DIM = 64
HIDDEN = 128
B, S = 1024, 20
N = B * S
NC, NS = 2, 16
NW = NC * NS
ROWS_PER_W = N // NW
R_CHUNK = 4
IDX_PER_CHUNK = R_CHUNK * S
N_CHUNKS = ROWS_PER_W // R_CHUNK
NBUF = 4


def _sc_gather_mean(idx2d, table):
    mesh = plsc.VectorSubcoreMesh(core_axis_name="c", subcore_axis_name="s")

    @functools.partial(
        pl.kernel,
        out_type=jax.ShapeDtypeStruct((N, EMBED_DIM // 2), jnp.uint32),
        mesh=mesh,
        scratch_types=[
            pltpu.VMEM((N_CHUNKS, IDX_PER_CHUNK), jnp.int32),
            [pltpu.VMEM((IDX_PER_CHUNK, EMBED_DIM // 2), jnp.uint32)] * NBUF,
            [pltpu.VMEM((R_CHUNK, EMBED_DIM // 2), jnp.uint32)] * NBUF,
            [pltpu.SemaphoreType.DMA] * NBUF,
            [pltpu.SemaphoreType.DMA] * NBUF,
            pltpu.VMEM_SHARED((NUM_RELATIONS, EMBED_DIM // 2), jnp.uint32),
        ],
        compiler_params=pltpu.CompilerParams(use_tc_tiling_on_sc=False),
    )
    def k(idx_hbm, table_hbm, agg_hbm, idx_v, rows_v, out_v, gsem, osem,
          tab_sh):
        wid = lax.axis_index("s") * NC + lax.axis_index("c")
        chunk0 = wid * N_CHUNKS

        @pl.when(lax.axis_index("s") == 0)
        def _():
            pltpu.sync_copy(table_hbm, tab_sh)

        pltpu.sync_copy(idx_hbm.at[pl.ds(chunk0 * 1, N_CHUNKS)], idx_v)
        plsc.subcore_barrier()

        def gather(g, b):
            pltpu.make_async_copy(
                tab_sh.at[idx_v.at[g]], rows_v[b], gsem[b]
            ).start()

        def gather_wait(b):
            pltpu.make_async_copy(
                tab_sh.at[idx_v.at[0]], rows_v[b], gsem[b]
            ).wait()

        def out_start(g, b):
            pltpu.make_async_copy(
                out_v[b], agg_hbm.at[pl.ds((chunk0 + g) * R_CHUNK, R_CHUNK)],
                osem[b],
            ).start()

        def out_wait(b):
            pltpu.make_async_copy(
                out_v[b], agg_hbm.at[pl.ds(0, R_CHUNK)], osem[b]
            ).wait()

        for b in range(NBUF):
            gather(b, b)

        def outer(t, carry):
            for b in range(NBUF):
                g = t * NBUF + b
                gather_wait(b)
                @pl.when(g >= NBUF)
                def _():
                    out_wait(b)

                rv, ov = rows_v[b], out_v[b]
                hi_mask = jnp.uint32(0xFFFF0000)
                for rr in range(R_CHUNK):
                    for c in range(EMBED_DIM // 32):
                        acc_a = jnp.zeros((16,), jnp.float32)
                        acc_b = jnp.zeros((16,), jnp.float32)
                        for j in range(S):
                            u = rv[rr * S + j, pl.ds(c * 16, 16)]
                            a = lax.bitcast_convert_type(u << 16, jnp.float32)
                            bb = lax.bitcast_convert_type(u & hi_mask, jnp.float32)
                            acc_a = acc_a + a
                            acc_b = acc_b + bb
                        ua = lax.bitcast_convert_type(acc_a * (1.0 / S), jnp.uint32)
                        ub = lax.bitcast_convert_type(acc_b * (1.0 / S), jnp.uint32)
                        ov[rr, pl.ds(c * 16, 16)] = (ua >> 16) | (ub & hi_mask)
                out_start(g, b)

                @pl.when(g + NBUF < N_CHUNKS)
                def _():
                    gather(g + NBUF, b)

            return carry

        lax.fori_loop(0, N_CHUNKS // NBUF, outer, None)
        for b in range(NBUF):
            out_wait(b)

    return k(idx2d, table)


def _mlp(agg_u32, W1e, W1o, b1, W2, b2):
    ROWS_BLK = 2048

    def body(a_ref, w1e_ref, w1o_ref, b1_ref, w2_ref, b2_ref, o_ref):
        u = a_ref[...]
        a_even = lax.bitcast_convert_type(u << 16, jnp.float32)
        a_odd = lax.bitcast_convert_type(u & jnp.uint32(0xFFFF0000),
                                         jnp.float32)
        h = (
            jnp.dot(a_even, w1e_ref[...], preferred_element_type=jnp.float32)
            + jnp.dot(a_odd, w1o_ref[...], preferred_element_type=jnp.float32)
        )
        h = jnp.maximum(h + b1_ref[...], 0.0)
        o_ref[...] = (
            jnp.dot(h, w2_ref[...], preferred_element_type=jnp.float32)
            + b2_ref[...]
        )

    return pl.pallas_call(
        body,
        grid=(N // ROWS_BLK,),
        in_specs=[
            pl.BlockSpec((ROWS_BLK, EMBED_DIM // 2), lambda i: (i, 0)),
            pl.BlockSpec((EMBED_DIM // 2, HIDDEN), lambda i: (0, 0)),
            pl.BlockSpec((EMBED_DIM // 2, HIDDEN), lambda i: (0, 0)),
            pl.BlockSpec((1, HIDDEN), lambda i: (0, 0)),
            pl.BlockSpec((HIDDEN, HIDDEN), lambda i: (0, 0)),
            pl.BlockSpec((1, HIDDEN), lambda i: (0, 0)),
        ],
        out_specs=pl.BlockSpec((ROWS_BLK, HIDDEN), lambda i: (i, 0)),
        out_shape=jax.ShapeDtypeStruct((N, HIDDEN), jnp.float32),
    )(agg_u32, W1e, W1o, b1, W2, b2)


def kernel(kg_spatial_matrix, rel_table, W1, b1, W2, b2):
    table_bf = rel_table.at[0].set(0.0).astype(jnp.bfloat16)
    table_u32 = jax.lax.bitcast_convert_type(
        table_bf.reshape(NUM_RELATIONS, EMBED_DIM // 2, 2), jnp.uint32
    )
    idx2d = kg_spatial_matrix.reshape(N // R_CHUNK, IDX_PER_CHUNK)
    agg_u32 = _sc_gather_mean(idx2d, table_u32)
    out = _mlp(agg_u32, W1[0::2], W1[1::2], b1.reshape(1, HIDDEN), W2,
               b2.reshape(1, HIDDEN))
    return out.reshape(B, S, HIDDEN)


# speedup vs baseline: 2.2912x; 1.1743x over previous
import functools

import jax
import jax.numpy as jnp
from jax import lax
from jax.experimental import pallas as pl
from jax.experimental.pallas import tpu as pltpu
from jax.experimental.pallas import tpu_sc as plsc

NUM_RELATIONS = 1000
EMBED_DIM = 64
HIDDEN = 128
B, S = 1024, 20
N = B * S
NC, NS = 2, 16
NW = NC * NS
ROWS_PER_W = N // NW
R_CHUNK = 4
IDX_PER_CHUNK = R_CHUNK * S
N_CHUNKS = ROWS_PER_W // R_CHUNK
NBUF = 4
PK = EMBED_DIM // 2


def _sc_gather_mean(idx_flat, table_u32):
    mesh = plsc.VectorSubcoreMesh(core_axis_name="c", subcore_axis_name="s")

    @functools.partial(
        pl.kernel,
        out_type=jax.ShapeDtypeStruct((N // R_CHUNK, R_CHUNK * PK),
                                      jnp.uint32),
        mesh=mesh,
        scratch_types=[
            pltpu.VMEM((ROWS_PER_W * S,), jnp.int32),
            [pltpu.VMEM((IDX_PER_CHUNK, PK), jnp.uint32)] * NBUF,
            [pltpu.VMEM((1, R_CHUNK * PK), jnp.uint32)] * NBUF,
            [pltpu.SemaphoreType.DMA] * NBUF,
            [pltpu.SemaphoreType.DMA] * NBUF,
            pltpu.VMEM_SHARED((NUM_RELATIONS, PK), jnp.uint32),
        ],
        compiler_params=pltpu.CompilerParams(use_tc_tiling_on_sc=False),
    )
    def k(idx_hbm, table_hbm, agg_hbm, idx_v, rows_v, out_v, gsem, osem,
          tab_sh):
        wid = lax.axis_index("s") * NC + lax.axis_index("c")
        chunk0 = wid * N_CHUNKS

        @pl.when(lax.axis_index("s") == 0)
        def _():
            pltpu.sync_copy(table_hbm, tab_sh)

        pltpu.sync_copy(idx_hbm.at[pl.ds(wid * ROWS_PER_W * S,
                                         ROWS_PER_W * S)], idx_v)
        plsc.subcore_barrier()

        def gather(g, b):
            pltpu.make_async_copy(
                tab_sh.at[idx_v.at[pl.ds(g * IDX_PER_CHUNK, IDX_PER_CHUNK)]],
                rows_v[b], gsem[b]
            ).start()

        def gather_wait(b):
            pltpu.make_async_copy(
                tab_sh.at[idx_v.at[pl.ds(0, IDX_PER_CHUNK)]],
                rows_v[b], gsem[b]
            ).wait()

        def out_start(g, b):
            pltpu.make_async_copy(
                out_v[b], agg_hbm.at[pl.ds(chunk0 + g, 1)], osem[b]
            ).start()

        def out_wait(b):
            pltpu.make_async_copy(
                out_v[b], agg_hbm.at[pl.ds(0, 1)], osem[b]
            ).wait()

        for b in range(NBUF):
            gather(b, b)

        def outer(t, carry):
            for b in range(NBUF):
                g = t * NBUF + b
                gather_wait(b)
                @pl.when(g >= NBUF)
                def _():
                    out_wait(b)

                rv, ov = rows_v[b], out_v[b]
                hi_mask = jnp.uint32(0xFFFF0000)
                for rr in range(R_CHUNK):
                    for c in range(PK // 16):
                        acc_a = jnp.zeros((16,), jnp.float32)
                        acc_b = jnp.zeros((16,), jnp.float32)
                        for j in range(S):
                            u = rv[rr * S + j, pl.ds(c * 16, 16)]
                            a = lax.bitcast_convert_type(u << 16, jnp.float32)
                            bb = lax.bitcast_convert_type(u & hi_mask,
                                                          jnp.float32)
                            acc_a = acc_a + a
                            acc_b = acc_b + bb
                        ua = lax.bitcast_convert_type(acc_a * (1.0 / S),
                                                      jnp.uint32)
                        ub = lax.bitcast_convert_type(acc_b * (1.0 / S),
                                                      jnp.uint32)
                        ov[0, pl.ds(rr * PK + c * 16, 16)] = (
                            (ua >> 16) | (ub & hi_mask)
                        )
                out_start(g, b)

                @pl.when(g + NBUF < N_CHUNKS)
                def _():
                    gather(g + NBUF, b)

            return carry

        lax.fori_loop(0, N_CHUNKS // NBUF, outer, None)
        for b in range(NBUF):
            out_wait(b)

    return k(idx_flat, table_u32)


BATCH_BLK = 64
POOL_BLK = BATCH_BLK * S
U_BLK = POOL_BLK // R_CHUNK


def _mlp(agg_u32, W1e_cat, W1o_cat, b1_cat, W2, b2):
    def body(a_ref, w1e_ref, w1o_ref, b1_ref, w2_ref, b2_ref, o_ref):
        u = a_ref[...]
        a_even = lax.bitcast_convert_type(u << 16, jnp.float32)
        a_odd = lax.bitcast_convert_type(u & jnp.uint32(0xFFFF0000),
                                         jnp.float32)
        h = (
            jnp.dot(a_even, w1e_ref[...], preferred_element_type=jnp.float32)
            + jnp.dot(a_odd, w1o_ref[...], preferred_element_type=jnp.float32)
        )
        h = jnp.maximum(h + b1_ref[...], 0.0)
        outs = []
        for kk in range(R_CHUNK):
            hk = h[:, kk * HIDDEN:(kk + 1) * HIDDEN]
            outs.append(
                jnp.dot(hk, w2_ref[...], preferred_element_type=jnp.float32)
                + b2_ref[...]
            )
        o = jnp.stack(outs, axis=1).reshape(POOL_BLK, HIDDEN)
        o_ref[...] = o.reshape(BATCH_BLK, S, HIDDEN)

    return pl.pallas_call(
        body,
        grid=(B // BATCH_BLK,),
        in_specs=[
            pl.BlockSpec((U_BLK, R_CHUNK * PK), lambda i: (i, 0)),
            pl.BlockSpec((HIDDEN, R_CHUNK * HIDDEN), lambda i: (0, 0)),
            pl.BlockSpec((HIDDEN, R_CHUNK * HIDDEN), lambda i: (0, 0)),
            pl.BlockSpec((1, R_CHUNK * HIDDEN), lambda i: (0, 0)),
            pl.BlockSpec((HIDDEN, HIDDEN), lambda i: (0, 0)),
            pl.BlockSpec((1, HIDDEN), lambda i: (0, 0)),
        ],
        out_specs=pl.BlockSpec((BATCH_BLK, S, HIDDEN), lambda i: (i, 0, 0)),
        out_shape=jax.ShapeDtypeStruct((B, S, HIDDEN), jnp.float32),
    )(agg_u32, W1e_cat, W1o_cat, b1_cat, W2, b2)


def kernel(kg_spatial_matrix, rel_table, W1, b1, W2, b2):
    table_bf = rel_table.at[0].set(0.0).astype(jnp.bfloat16)
    table_u32 = jax.lax.bitcast_convert_type(
        table_bf.reshape(NUM_RELATIONS, PK, 2), jnp.uint32
    )
    idx_flat = kg_spatial_matrix.reshape(-1)
    agg_u32 = _sc_gather_mean(idx_flat, table_u32)

    eye4 = jnp.eye(R_CHUNK, dtype=W1.dtype)
    W1e_cat = jnp.einsum("kl,ch->kclh", eye4, W1[0::2]).reshape(
        R_CHUNK * PK, R_CHUNK * HIDDEN
    )
    W1o_cat = jnp.einsum("kl,ch->kclh", eye4, W1[1::2]).reshape(
        R_CHUNK * PK, R_CHUNK * HIDDEN
    )
    b1_cat = jnp.tile(b1, R_CHUNK).reshape(1, R_CHUNK * HIDDEN)
    return _mlp(agg_u32, W1e_cat, W1o_cat, b1_cat, W2,
                b2.reshape(1, HIDDEN))


# speedup vs baseline: 2.2957x; 1.0019x over previous
import functools

import jax
import jax.numpy as jnp
from jax import lax
from jax.experimental import pallas as pl
from jax.experimental.pallas import tpu as pltpu
from jax.experimental.pallas import tpu_sc as plsc

NUM_RELATIONS = 1000
EMBED_DIM = 64
HIDDEN = 128
B, S = 1024, 20
N = B * S
NC, NS = 2, 16
NW = NC * NS
PK = EMBED_DIM // 2
QROWS = N // 4
Q_PER_W = QROWS // NW
Q_CHUNK = 2
N_CHUNKS = Q_PER_W // Q_CHUNK
IDX_PER_K = Q_CHUNK * S
NBUF = 2


def _sc_gather_mean(idx_flat, table_u32):
    mesh = plsc.VectorSubcoreMesh(core_axis_name="c", subcore_axis_name="s")

    @functools.partial(
        pl.kernel,
        out_type=jax.ShapeDtypeStruct((QROWS, 4 * PK), jnp.uint32),
        mesh=mesh,
        scratch_types=[
            pltpu.VMEM((4, Q_PER_W * S), jnp.int32),
            [pltpu.VMEM((4 * IDX_PER_K, PK), jnp.uint32)] * NBUF,
            [pltpu.VMEM((Q_CHUNK, 4 * PK), jnp.uint32)] * NBUF,
            [pltpu.SemaphoreType.DMA] * NBUF,
            [pltpu.SemaphoreType.DMA] * NBUF,
            pltpu.VMEM_SHARED((NUM_RELATIONS, PK), jnp.uint32),
        ],
        compiler_params=pltpu.CompilerParams(use_tc_tiling_on_sc=False),
    )
    def k(idx_hbm, table_hbm, agg_hbm, idx_v, rows_v, out_v, gsem, osem,
          tab_sh):
        wid = lax.axis_index("s") * NC + lax.axis_index("c")
        q0 = wid * Q_PER_W

        @pl.when(lax.axis_index("s") == 0)
        def _():
            pltpu.sync_copy(table_hbm, tab_sh)

        for kk in range(4):
            pltpu.sync_copy(
                idx_hbm.at[pl.ds(kk * QROWS * S + q0 * S, Q_PER_W * S)],
                idx_v.at[kk],
            )
        plsc.subcore_barrier()

        def gather(t, b):
            for kk in range(4):
                pltpu.make_async_copy(
                    tab_sh.at[idx_v.at[kk, pl.ds(t * IDX_PER_K, IDX_PER_K)]],
                    rows_v[b].at[pl.ds(kk * IDX_PER_K, IDX_PER_K)],
                    gsem[b],
                ).start()

        def gather_wait(b):
            for kk in range(4):
                pltpu.make_async_copy(
                    tab_sh.at[idx_v.at[0, pl.ds(0, IDX_PER_K)]],
                    rows_v[b].at[pl.ds(kk * IDX_PER_K, IDX_PER_K)],
                    gsem[b],
                ).wait()

        def out_start(t, b):
            pltpu.make_async_copy(
                out_v[b], agg_hbm.at[pl.ds(q0 + t * Q_CHUNK, Q_CHUNK)],
                osem[b],
            ).start()

        def out_wait(b):
            pltpu.make_async_copy(
                out_v[b], agg_hbm.at[pl.ds(0, Q_CHUNK)], osem[b]
            ).wait()

        for b in range(NBUF):
            gather(b, b)

        def outer(tt, carry):
            for b in range(NBUF):
                t = tt * NBUF + b
                gather_wait(b)
                @pl.when(t >= NBUF)
                def _():
                    out_wait(b)

                rv, ov = rows_v[b], out_v[b]
                hi_mask = jnp.uint32(0xFFFF0000)
                for p in range(Q_CHUNK):
                    for kk in range(4):
                        for c in range(PK // 16):
                            acc_a = jnp.zeros((16,), jnp.float32)
                            acc_b = jnp.zeros((16,), jnp.float32)
                            for j in range(S):
                                u = rv[kk * IDX_PER_K + p * S + j,
                                       pl.ds(c * 16, 16)]
                                a = lax.bitcast_convert_type(
                                    u << 16, jnp.float32)
                                bb = lax.bitcast_convert_type(
                                    u & hi_mask, jnp.float32)
                                acc_a = acc_a + a
                                acc_b = acc_b + bb
                            ua = lax.bitcast_convert_type(
                                acc_a * (1.0 / S), jnp.uint32)
                            ub = lax.bitcast_convert_type(
                                acc_b * (1.0 / S), jnp.uint32)
                            ov[p, pl.ds(kk * PK + c * 16, 16)] = (
                                (ua >> 16) | (ub & hi_mask)
                            )
                out_start(t, b)

                @pl.when(t + NBUF < N_CHUNKS)
                def _():
                    gather(t + NBUF, b)

            return carry

        lax.fori_loop(0, N_CHUNKS // NBUF, outer, None)
        for b in range(NBUF):
            out_wait(b)

    return k(idx_flat, table_u32)


GRID_I = 4
U_BLK = QROWS // GRID_I
BATCH_BLK = U_BLK // S


def _mlp(agg_u32, W1e_cat, W1o_cat, b1, W2, b2):
    def body(a_ref, w1e_ref, w1o_ref, b1_ref, w2_ref, b2_ref, o_ref,
             ae_s, ao_s):
        @pl.when(pl.program_id(1) == 0)
        def _():
            u = a_ref[...]
            ae_s[...] = lax.bitcast_convert_type(u << 16, jnp.float32)
            ao_s[...] = lax.bitcast_convert_type(
                u & jnp.uint32(0xFFFF0000), jnp.float32)

        h = (
            jnp.dot(ae_s[...], w1e_ref[...],
                    preferred_element_type=jnp.float32)
            + jnp.dot(ao_s[...], w1o_ref[...],
                      preferred_element_type=jnp.float32)
        )
        h = jnp.maximum(h + b1_ref[...], 0.0)
        o = (
            jnp.dot(h, w2_ref[...], preferred_element_type=jnp.float32)
            + b2_ref[...]
        )
        o_ref[...] = o.reshape(BATCH_BLK, S, HIDDEN)

    return pl.pallas_call(
        body,
        grid=(GRID_I, 4),
        in_specs=[
            pl.BlockSpec((U_BLK, 4 * PK), lambda i, k: (i, 0)),
            pl.BlockSpec((4 * PK, HIDDEN), lambda i, k: (0, k)),
            pl.BlockSpec((4 * PK, HIDDEN), lambda i, k: (0, k)),
            pl.BlockSpec((1, HIDDEN), lambda i, k: (0, 0)),
            pl.BlockSpec((HIDDEN, HIDDEN), lambda i, k: (0, 0)),
            pl.BlockSpec((1, HIDDEN), lambda i, k: (0, 0)),
        ],
        out_specs=pl.BlockSpec((BATCH_BLK, S, HIDDEN),
                               lambda i, k: (4 * k + i, 0, 0)),
        out_shape=jax.ShapeDtypeStruct((B, S, HIDDEN), jnp.float32),
        scratch_shapes=[
            pltpu.VMEM((U_BLK, 4 * PK), jnp.float32),
            pltpu.VMEM((U_BLK, 4 * PK), jnp.float32),
        ],
    )(agg_u32, W1e_cat, W1o_cat, b1, W2, b2)


def kernel(kg_spatial_matrix, rel_table, W1, b1, W2, b2):
    table_bf = rel_table.at[0].set(0.0).astype(jnp.bfloat16)
    table_u32 = jax.lax.bitcast_convert_type(
        table_bf.reshape(NUM_RELATIONS, PK, 2), jnp.uint32
    )
    idx_flat = kg_spatial_matrix.reshape(-1)
    agg_u32 = _sc_gather_mean(idx_flat, table_u32)

    eye4 = jnp.eye(4, dtype=W1.dtype)
    W1e_cat = jnp.einsum("kl,ch->kclh", eye4, W1[0::2]).reshape(
        4 * PK, 4 * HIDDEN
    )
    W1o_cat = jnp.einsum("kl,ch->kclh", eye4, W1[1::2]).reshape(
        4 * PK, 4 * HIDDEN
    )
    return _mlp(agg_u32, W1e_cat, W1o_cat, b1.reshape(1, HIDDEN), W2,
                b2.reshape(1, HIDDEN))


# speedup vs baseline: 2.4158x; 1.0523x over previous
import functools

import jax
import jax.numpy as jnp
from jax import lax
from jax.experimental import pallas as pl
from jax.experimental.pallas import tpu as pltpu
from jax.experimental.pallas import tpu_sc as plsc

NUM_RELATIONS = 1000
EMBED_DIM = 64
HIDDEN = 128
B, S = 1024, 20
N = B * S
NC, NS = 2, 16
NW = NC * NS
PK = EMBED_DIM // 2
QROWS = N // 4
Q_PER_W = QROWS // NW
Q_CHUNK = 2
N_CHUNKS = Q_PER_W // Q_CHUNK
IDX_PER_K = Q_CHUNK * S
NBUF = 2


def _sc_gather_mean(idx_flat, table_u32):
    mesh = plsc.VectorSubcoreMesh(core_axis_name="c", subcore_axis_name="s")

    @functools.partial(
        pl.kernel,
        out_type=jax.ShapeDtypeStruct((QROWS, 4 * PK), jnp.uint32),
        mesh=mesh,
        scratch_types=[
            pltpu.VMEM((4, Q_PER_W * S), jnp.int32),
            [pltpu.VMEM((4 * IDX_PER_K, PK), jnp.uint32)] * NBUF,
            [pltpu.VMEM((Q_CHUNK, 4 * PK), jnp.uint32)] * NBUF,
            [pltpu.SemaphoreType.DMA] * NBUF,
            [pltpu.SemaphoreType.DMA] * NBUF,
            pltpu.VMEM_SHARED((NUM_RELATIONS, PK), jnp.uint32),
        ],
        compiler_params=pltpu.CompilerParams(use_tc_tiling_on_sc=False),
    )
    def k(idx_hbm, table_hbm, agg_hbm, idx_v, rows_v, out_v, gsem, osem,
          tab_sh):
        wid = lax.axis_index("s") * NC + lax.axis_index("c")
        q0 = wid * Q_PER_W

        @pl.when(lax.axis_index("s") == 0)
        def _():
            pltpu.sync_copy(table_hbm, tab_sh)

        for kk in range(4):
            pltpu.sync_copy(
                idx_hbm.at[pl.ds(kk * QROWS * S + q0 * S, Q_PER_W * S)],
                idx_v.at[kk],
            )
        plsc.subcore_barrier()

        def gather(t, b):
            for kk in range(4):
                pltpu.make_async_copy(
                    tab_sh.at[idx_v.at[kk, pl.ds(t * IDX_PER_K, IDX_PER_K)]],
                    rows_v[b].at[pl.ds(kk * IDX_PER_K, IDX_PER_K)],
                    gsem[b],
                ).start()

        def gather_wait(b):
            for kk in range(4):
                pltpu.make_async_copy(
                    tab_sh.at[idx_v.at[0, pl.ds(0, IDX_PER_K)]],
                    rows_v[b].at[pl.ds(kk * IDX_PER_K, IDX_PER_K)],
                    gsem[b],
                ).wait()

        def out_start(t, b):
            pltpu.make_async_copy(
                out_v[b], agg_hbm.at[pl.ds(q0 + t * Q_CHUNK, Q_CHUNK)],
                osem[b],
            ).start()

        def out_wait(b):
            pltpu.make_async_copy(
                out_v[b], agg_hbm.at[pl.ds(0, Q_CHUNK)], osem[b]
            ).wait()

        for b in range(NBUF):
            gather(b, b)

        def outer(tt, carry):
            for b in range(NBUF):
                t = tt * NBUF + b
                gather_wait(b)
                @pl.when(t >= NBUF)
                def _():
                    out_wait(b)

                rv, ov = rows_v[b], out_v[b]
                hi_mask = jnp.uint32(0xFFFF0000)
                for p in range(Q_CHUNK):
                    for kk in range(4):
                        for c in range(PK // 16):
                            acc_a = jnp.zeros((16,), jnp.float32)
                            acc_b = jnp.zeros((16,), jnp.float32)
                            for j in range(S):
                                u = rv[kk * IDX_PER_K + p * S + j,
                                       pl.ds(c * 16, 16)]
                                a = lax.bitcast_convert_type(
                                    u << 16, jnp.float32)
                                bb = lax.bitcast_convert_type(
                                    u & hi_mask, jnp.float32)
                                acc_a = acc_a + a
                                acc_b = acc_b + bb
                            ua = lax.bitcast_convert_type(
                                acc_a * (1.0 / S), jnp.uint32)
                            ub = lax.bitcast_convert_type(
                                acc_b * (1.0 / S), jnp.uint32)
                            ov[p, pl.ds(kk * PK + c * 16, 16)] = (
                                (ua >> 16) | (ub & hi_mask)
                            )
                out_start(t, b)

                @pl.when(t + NBUF < N_CHUNKS)
                def _():
                    gather(t + NBUF, b)

            return carry

        lax.fori_loop(0, N_CHUNKS // NBUF, outer, None)
        for b in range(NBUF):
            out_wait(b)

    return k(idx_flat, table_u32)


GRID_I = 4
U_BLK = QROWS // GRID_I
BATCH_BLK = U_BLK // S


def _mlp(agg_u32, W1e, W1o, b1, W2, b2):
    def body(a_ref, w1e_ref, w1o_ref, b1_ref, w2_ref, b2_ref, o_ref):
        u = a_ref[...]
        a_even = lax.bitcast_convert_type(u << 16, jnp.float32)
        a_odd = lax.bitcast_convert_type(u & jnp.uint32(0xFFFF0000),
                                         jnp.float32)
        for kk in range(4):
            ek = a_even[:, kk * PK:(kk + 1) * PK]
            ok = a_odd[:, kk * PK:(kk + 1) * PK]
            h = (
                jnp.dot(ek, w1e_ref[...], preferred_element_type=jnp.float32)
                + jnp.dot(ok, w1o_ref[...],
                          preferred_element_type=jnp.float32)
            )
            h = jnp.maximum(h + b1_ref[...], 0.0)
            o = (
                jnp.dot(h, w2_ref[...], preferred_element_type=jnp.float32)
                + b2_ref[...]
            )
            o_ref[kk] = o.reshape(BATCH_BLK, S, HIDDEN)

    return pl.pallas_call(
        body,
        grid=(GRID_I,),
        in_specs=[
            pl.BlockSpec((U_BLK, 4 * PK), lambda i: (i, 0)),
            pl.BlockSpec((PK, HIDDEN), lambda i: (0, 0)),
            pl.BlockSpec((PK, HIDDEN), lambda i: (0, 0)),
            pl.BlockSpec((1, HIDDEN), lambda i: (0, 0)),
            pl.BlockSpec((HIDDEN, HIDDEN), lambda i: (0, 0)),
            pl.BlockSpec((1, HIDDEN), lambda i: (0, 0)),
        ],
        out_specs=pl.BlockSpec((4, BATCH_BLK, S, HIDDEN),
                               lambda i: (0, i, 0, 0)),
        out_shape=jax.ShapeDtypeStruct((4, B // 4, S, HIDDEN), jnp.float32),
    )(agg_u32, W1e, W1o, b1, W2, b2)


def kernel(kg_spatial_matrix, rel_table, W1, b1, W2, b2):
    table_bf = rel_table.at[0].set(0.0).astype(jnp.bfloat16)
    table_u32 = jax.lax.bitcast_convert_type(
        table_bf.reshape(NUM_RELATIONS, PK, 2), jnp.uint32
    )
    idx_flat = kg_spatial_matrix.reshape(-1)
    agg_u32 = _sc_gather_mean(idx_flat, table_u32)
    out4 = _mlp(agg_u32, W1[0::2], W1[1::2], b1.reshape(1, HIDDEN), W2,
                b2.reshape(1, HIDDEN))
    return out4.reshape(B, S, HIDDEN)
